# Initial kernel scaffold; baseline (speedup 1.0000x reference)
import jax
import jax.numpy as jnp
from jax.experimental import pallas as pl


def kernel(x, adj_indices, adj_values):
    raise NotImplementedError("write your pallas kernel here")


# speedup vs baseline: 2.6194x; 2.6194x over previous
import functools

import jax
import jax.numpy as jnp
from jax import lax
from jax.experimental import pallas as pl
from jax.experimental.pallas import tpu as pltpu
from jax.experimental.pallas import tpu_sc as plsc

_N = 10000
_E = 160000
_D = 256
_DH = _D // 2
_K = 128
_NS = 16
_NC = 2
_CPT = 80
_EPAD = _CPT * _NS * _K
_RPT = 624


def _sc_spmm_body(xr_hbm, g0_hbm, g1_hbm, row_hbm, val_hbm, out_hbm,
                  acc, gidx_v, row_v, val_v, rows_v, sem):
    c = lax.axis_index("c")
    s = lax.axis_index("s")

    @pl.loop(0, _K)
    def _zero(e):
        for d in range(_DH // 16):
            rows_v[e, pl.ds(d * 16, 16)] = jnp.zeros((16,), jnp.float32)

    @pl.loop(0, 4)
    def _zinit(i):
        pltpu.sync_copy(rows_v,
                        acc.at[pl.ds(s * _RPT + i * _K, _K)])

    @pl.when(s < 15)
    def _():
        pltpu.sync_copy(rows_v.at[pl.ds(0, 112)],
                        acc.at[pl.ds(s * _RPT + 4 * _K, 112)])

    @pl.when(s == 15)
    def _():
        pltpu.sync_copy(rows_v, acc.at[pl.ds(15 * _RPT + 4 * _K, _K)])

    base = s * _CPT

    @pl.when(c == 0)
    def _():
        pltpu.sync_copy(g0_hbm.at[pl.ds(base, _CPT)], gidx_v)

    @pl.when(c == 1)
    def _():
        pltpu.sync_copy(g1_hbm.at[pl.ds(base, _CPT)], gidx_v)

    pltpu.sync_copy(row_hbm.at[pl.ds(base, _CPT)], row_v)
    pltpu.sync_copy(val_hbm.at[pl.ds(base, _CPT)], val_v)

    plsc.subcore_barrier()

    @pl.loop(0, _CPT)
    def _chunk(ci):
        pltpu.async_copy(xr_hbm.at[gidx_v.at[ci]], rows_v, sem).wait()

        @pl.loop(0, _K // 16)
        def _scale(g):
            val16 = val_v[ci, pl.ds(g * 16, 16)]
            for j in range(16):
                e = g * 16 + j
                vb = jnp.full((16,), val16[j], jnp.float32)
                for d in range(_DH // 16):
                    sl = pl.ds(d * 16, 16)
                    rows_v[e, sl] = rows_v[e, sl] * vb

        pltpu.sync_copy(rows_v, acc.at[row_v.at[ci]], add=True)

    plsc.subcore_barrier()

    @pl.when(s < 15)
    def _():
        pltpu.sync_copy(acc.at[pl.ds(s * _RPT, _RPT)],
                        out_hbm.at[c, pl.ds(s * _RPT, _RPT)])

    @pl.when(s == 15)
    def _():
        pltpu.sync_copy(acc.at[pl.ds(15 * _RPT, 640)],
                        out_hbm.at[c, pl.ds(15 * _RPT, 640)])


@jax.jit
def _sc_spmm(xr, g0, g1, row2d, val2d):
    mesh = plsc.VectorSubcoreMesh(core_axis_name="c", subcore_axis_name="s")
    fn = pl.kernel(
        _sc_spmm_body,
        out_type=jax.ShapeDtypeStruct((_NC, _N, _DH), jnp.float32),
        mesh=mesh,
        scratch_types=[
            pltpu.VMEM_SHARED((_N, _DH), jnp.float32),
            pltpu.VMEM((_CPT, _K), jnp.int32),
            pltpu.VMEM((_CPT, _K), jnp.int32),
            pltpu.VMEM((_CPT, _K), jnp.float32),
            pltpu.VMEM((_K, _DH), jnp.float32),
            pltpu.SemaphoreType.DMA,
        ],
    )
    return fn(xr, g0, g1, row2d, val2d)


def _tc_norm_body(sum_ref, o_ref):
    a = sum_ref[0]
    b = sum_ref[1]
    sq = (jnp.sum(a * a, axis=1) + jnp.sum(b * b, axis=1)
          - 2.0 * a[:, 0] * a[:, 0])
    coeff = 1.0 / jnp.sqrt(jnp.abs(sq))
    o_ref[:, : _DH] = a * coeff[:, None]
    o_ref[:, _DH:] = b * coeff[:, None]


@jax.jit
def _tc_norm(sums):
    blk = 2000
    return pl.pallas_call(
        _tc_norm_body,
        grid=(_N // blk,),
        in_specs=[pl.BlockSpec((_NC, blk, _DH), lambda i: (0, i, 0))],
        out_specs=pl.BlockSpec((blk, _D), lambda i: (i, 0)),
        out_shape=jax.ShapeDtypeStruct((_N, _D), jnp.float32),
    )(sums)


def kernel(x, adj_indices, adj_values):
    row = adj_indices[0]
    col = adj_indices[1]
    pad = _EPAD - _E
    shape2d = (_CPT * _NS, _K)
    row2d = jnp.pad(row, (0, pad)).reshape(shape2d)
    val2d = jnp.pad(adj_values, (0, pad)).reshape(shape2d)
    g0 = jnp.pad(col * 2, (0, pad)).reshape(shape2d)
    g1 = jnp.pad(col * 2 + 1, (0, pad)).reshape(shape2d)
    xr = x.reshape(2 * _N, _DH)
    sums = _sc_spmm(xr, g0, g1, row2d, val2d)
    return _tc_norm(sums)
